# Initial kernel scaffold; baseline (speedup 1.0000x reference)
import jax
import jax.numpy as jnp
from jax.experimental import pallas as pl


def kernel(features, edge_index, Ws0, Wn0, b0, Ws1, Wn1, b1, Ws2, Wn2, b2):
    raise NotImplementedError("write your pallas kernel here")


# speedup vs baseline: 4.6765x; 4.6765x over previous
import jax
import jax.numpy as jnp
from jax import lax
from jax.experimental import pallas as pl
from jax.experimental.pallas import tpu as pltpu
from jax.experimental.pallas import tpu_sc as plsc

N_NODES = 10000
N_EDGES = 320000
NC = 2
NS = 16
NW = NC * NS
E_PER_TILE = N_EDGES // NW
CHUNK = 80
N_CHUNKS = E_PER_TILE // CHUNK
N_PAD = 10240
ROWS_PER_TILE = N_PAD // NS
F = 128

_MESH = plsc.VectorSubcoreMesh(core_axis_name="c", subcore_axis_name="s")


def _zero_fill(ref, nrows):
    def body(i, _):
        for j in range(F // 16):
            ref[i, pl.ds(j * 16, 16)] = jnp.zeros((16,), jnp.float32)
        return 0
    lax.fori_loop(0, nrows, body, 0)


def _sc_agg(h, src, dst):
    def body(h_hbm, src_hbm, dst_hbm, agg_out, agg_sh, src_v, dst_v, rows_v,
             sem):
        c = lax.axis_index("c")
        s = lax.axis_index("s")
        wid = s * NC + c

        _zero_fill(rows_v, CHUNK)
        for k in range(ROWS_PER_TILE // CHUNK):
            r0 = s * ROWS_PER_TILE + k * CHUNK
            pltpu.sync_copy(rows_v, agg_sh.at[pl.ds(r0, CHUNK)])
        plsc.subcore_barrier()

        def ebody(i, _):
            base = wid * E_PER_TILE + i * CHUNK
            pltpu.sync_copy(src_hbm.at[pl.ds(base, CHUNK)], src_v)
            pltpu.sync_copy(dst_hbm.at[pl.ds(base, CHUNK)], dst_v)
            pltpu.async_copy(h_hbm.at[src_v], rows_v, sem).wait()
            pltpu.sync_copy(rows_v, agg_sh.at[dst_v], add=True)
            return 0
        lax.fori_loop(0, N_CHUNKS, ebody, 0)
        plsc.subcore_barrier()

        for k in range(ROWS_PER_TILE // CHUNK):
            r0 = s * ROWS_PER_TILE + k * CHUNK
            pltpu.sync_copy(agg_sh.at[pl.ds(r0, CHUNK)], rows_v)
            pltpu.sync_copy(rows_v, agg_out.at[c, pl.ds(r0, CHUNK)])

    return pl.kernel(
        body,
        out_type=jax.ShapeDtypeStruct((NC, N_PAD, F), jnp.float32),
        mesh=_MESH,
        scratch_types=(
            pltpu.VMEM_SHARED((N_PAD, F), jnp.float32),
            pltpu.VMEM((CHUNK,), jnp.int32),
            pltpu.VMEM((CHUNK,), jnp.int32),
            pltpu.VMEM((CHUNK, F), jnp.float32),
            pltpu.SemaphoreType.DMA,
        ),
    )(h, src, dst)


def _sc_deg(dst):
    def body(dst_hbm, deg_out, deg_sh, dst_v, rows_v):
        c = lax.axis_index("c")
        s = lax.axis_index("s")
        wid = s * NC + c

        _zero_fill(rows_v, CHUNK)
        for k in range(ROWS_PER_TILE // CHUNK):
            r0 = s * ROWS_PER_TILE + k * CHUNK
            pltpu.sync_copy(rows_v, deg_sh.at[pl.ds(r0, CHUNK)])

        def ones_body(i, _):
            for j in range(F // 16):
                rows_v[i, pl.ds(j * 16, 16)] = jnp.ones((16,), jnp.float32)
            return 0
        lax.fori_loop(0, CHUNK, ones_body, 0)
        plsc.subcore_barrier()

        def ebody(i, _):
            base = wid * E_PER_TILE + i * CHUNK
            pltpu.sync_copy(dst_hbm.at[pl.ds(base, CHUNK)], dst_v)
            pltpu.sync_copy(rows_v, deg_sh.at[dst_v], add=True)
            return 0
        lax.fori_loop(0, N_CHUNKS, ebody, 0)
        plsc.subcore_barrier()

        for k in range(ROWS_PER_TILE // CHUNK):
            r0 = s * ROWS_PER_TILE + k * CHUNK
            pltpu.sync_copy(deg_sh.at[pl.ds(r0, CHUNK)], rows_v)
            pltpu.sync_copy(rows_v, deg_out.at[c, pl.ds(r0, CHUNK)])

    return pl.kernel(
        body,
        out_type=jax.ShapeDtypeStruct((NC, N_PAD, F), jnp.float32),
        mesh=_MESH,
        scratch_types=(
            pltpu.VMEM_SHARED((N_PAD, F), jnp.float32),
            pltpu.VMEM((CHUNK,), jnp.int32),
            pltpu.VMEM((CHUNK, F), jnp.float32),
        ),
    )(dst)


def _tc_layer(h, a0, a1, d0, d1, Ws, Wn, b, act):
    n, din = h.shape
    dout = Ws.shape[1]
    blk = 1000

    def body(h_ref, a0_ref, a1_ref, d0_ref, d1_ref, ws_ref, wn_ref, b_ref,
             o_ref):
        deg = d0_ref[:, 0:1] + d1_ref[:, 0:1]
        inv = 1.0 / jnp.maximum(deg, 1.0)
        hn = (a0_ref[...] + a1_ref[...]) * inv
        out = jnp.dot(h_ref[...], ws_ref[...],
                      preferred_element_type=jnp.float32)
        out = out + jnp.dot(hn, wn_ref[...],
                            preferred_element_type=jnp.float32)
        out = out + b_ref[...]
        if act:
            out = jnp.maximum(out, 0.0)
        o_ref[...] = out

    return pl.pallas_call(
        body,
        grid=(n // blk,),
        in_specs=[
            pl.BlockSpec((blk, din), lambda i: (i, 0)),
            pl.BlockSpec((blk, din), lambda i: (i, 0)),
            pl.BlockSpec((blk, din), lambda i: (i, 0)),
            pl.BlockSpec((blk, F), lambda i: (i, 0)),
            pl.BlockSpec((blk, F), lambda i: (i, 0)),
            pl.BlockSpec((din, dout), lambda i: (0, 0)),
            pl.BlockSpec((din, dout), lambda i: (0, 0)),
            pl.BlockSpec((1, dout), lambda i: (0, 0)),
        ],
        out_specs=pl.BlockSpec((blk, dout), lambda i: (i, 0)),
        out_shape=jax.ShapeDtypeStruct((n, dout), jnp.float32),
    )(h, a0, a1, d0, d1, Ws, Wn, b.reshape(1, dout))


def kernel(features, edge_index, Ws0, Wn0, b0, Ws1, Wn1, b1, Ws2, Wn2, b2):
    src = edge_index[0]
    dst = edge_index[1]

    degp = _sc_deg(dst)
    d0, d1 = degp[0], degp[1]
    agg0 = _sc_agg(features, src, dst)
    h1 = _tc_layer(features, agg0[0], agg0[1], d0, d1, Ws0, Wn0, b0, True)
    agg1 = _sc_agg(h1, src, dst)
    h2 = _tc_layer(h1, agg1[0], agg1[1], d0, d1, Ws1, Wn1, b1, True)
    agg2 = _sc_agg(h2, src, dst)
    return _tc_layer(h2, agg2[0], agg2[1], d0, d1, Ws2, Wn2, b2, False)
